# baseline (speedup 1.0000x reference)
import functools

import jax
import jax.numpy as jnp
from jax import lax
from jax.experimental import pallas as pl
from jax.experimental.pallas import tpu as pltpu
from jax.experimental.pallas import tpu_sc as plsc

N_STRUCTURES = 1024
TAB_PAD = 128
LANES = 16
ROW = 128


def _make_sc_partials(n_atoms):
    info = plsc.get_sparse_core_info()
    nc, ns = info.num_cores, info.num_subcores
    nw = nc * ns
    assert n_atoms % ROW == 0
    n_rows = n_atoms // ROW
    rows_w = n_rows // nw
    tail_rows = n_rows - rows_w * nw
    assert tail_rows <= nw

    chunk_rows = rows_w
    for cand in (100, 96, 90, 80, 78, 75, 65, 64, 60, 50, 40, 39, 32, 30,
                 26, 25, 20, 16, 15, 13, 10, 8, 6, 5, 4, 3, 2, 1):
        if rows_w % cand == 0:
            chunk_rows = cand
            break
    n_chunks = rows_w // chunk_rows
    vec_per_chunk = chunk_rows * (ROW // LANES)
    idx_len = ((chunk_rows + LANES - 1) // LANES) * LANES

    mesh = plsc.VectorSubcoreMesh(core_axis_name="c", subcore_axis_name="s")

    @functools.partial(
        pl.kernel,
        mesh=mesh,
        out_type=jax.ShapeDtypeStruct((nw, N_STRUCTURES), jnp.float32),
        compiler_params=pltpu.CompilerParams(
            needs_layout_passes=False, use_tc_tiling_on_sc=False
        ),
        scratch_types=[
            pltpu.VMEM((chunk_rows, ROW), jnp.float32),
            pltpu.VMEM((chunk_rows, ROW), jnp.int32),
            pltpu.VMEM((chunk_rows, ROW), jnp.int32),
            pltpu.VMEM((chunk_rows, ROW), jnp.float32),
            pltpu.VMEM((chunk_rows, ROW), jnp.int32),
            pltpu.VMEM((chunk_rows, ROW), jnp.int32),
            pltpu.VMEM((idx_len,), jnp.int32),
            pltpu.VMEM((idx_len,), jnp.int32),
            pltpu.VMEM((1, ROW), jnp.float32),
            pltpu.VMEM((1, ROW), jnp.int32),
            pltpu.VMEM((1, ROW), jnp.int32),
            pltpu.VMEM((TAB_PAD,), jnp.float32),
            pltpu.VMEM((TAB_PAD,), jnp.float32),
            pltpu.VMEM((N_STRUCTURES,), jnp.float32),
            pltpu.SemaphoreType.DMA,
            pltpu.SemaphoreType.DMA,
            pltpu.SemaphoreType.DMA,
        ],
    )
    def sc_kernel(e_hbm, z_hbm, b_hbm, scale_hbm, shift_hbm, out_hbm,
                  e0, z0, b0, e1, z1, b1, idx0, idx1, et, zt, bt,
                  scale_v, shift_v, acc_v, sem0, sem1, semt):
        sid = lax.axis_index("s")
        wid = sid * nc + lax.axis_index("c")
        row_base = wid * rows_w
        bufs = ((e0, z0, b0), (e1, z1, b1))
        idxs = (idx0, idx1)
        sems = (sem0, sem1)

        def start_chunk(ci, p):
            r0 = row_base + ci * chunk_rows
            for k in range(idx_len // LANES):
                idxs[p][pl.ds(k * LANES, LANES)] = (
                    lax.iota(jnp.int32, LANES) + (r0 + k * LANES)
                )
            rows = idxs[p].at[pl.ds(0, chunk_rows)]
            return (
                pltpu.async_copy(e_hbm.at[rows], bufs[p][0], sems[p]),
                pltpu.async_copy(z_hbm.at[rows], bufs[p][1], sems[p]),
                pltpu.async_copy(b_hbm.at[rows], bufs[p][2], sems[p]),
            )

        in_flight = start_chunk(0, 0)

        pltpu.sync_copy(scale_hbm, scale_v)
        pltpu.sync_copy(shift_hbm, shift_v)

        def zero_body(i, _):
            acc_v[pl.ds(i * LANES, LANES)] = jnp.zeros((LANES,), jnp.float32)
            return 0

        lax.fori_loop(0, N_STRUCTURES // LANES, zero_body, 0)

        def process(e_v, z_v, b_v, nvec):
            @plsc.parallel_loop(0, nvec, 1, unroll=8)
            def _(j):
                r = j // (ROW // LANES)
                c = (j % (ROW // LANES)) * LANES
                e = e_v[r, pl.ds(c, LANES)]
                z = z_v[r, pl.ds(c, LANES)]
                b = b_v[r, pl.ds(c, LANES)]
                sc = plsc.load_gather(scale_v, [z])
                sh = plsc.load_gather(shift_v, [z])
                plsc.addupdate_scatter(acc_v, [b], e * sc + sh)

        for ci in range(n_chunks):
            p = ci % 2
            for d in in_flight:
                d.wait()
            if ci + 1 < n_chunks:
                in_flight = start_chunk(ci + 1, (ci + 1) % 2)
            e_v, z_v, b_v = bufs[p]
            process(e_v, z_v, b_v, vec_per_chunk)

        if tail_rows:
            @pl.when(wid < tail_rows)
            def _():
                tr = nw * rows_w + wid
                idx0[pl.ds(0, LANES)] = lax.iota(jnp.int32, LANES) + tr
                rows = idx0.at[pl.ds(0, 1)]
                d1 = pltpu.async_copy(e_hbm.at[rows], et, semt)
                d2 = pltpu.async_copy(z_hbm.at[rows], zt, semt)
                d3 = pltpu.async_copy(b_hbm.at[rows], bt, semt)
                d1.wait()
                d2.wait()
                d3.wait()
                process(et, zt, bt, ROW // LANES)

        pltpu.sync_copy(acc_v, out_hbm.at[wid])

    return sc_kernel, nw


def _sum_partials_body(p_ref, o_ref):
    o_ref[:] = jnp.sum(p_ref[:], axis=0)


def kernel(local_energies, Z, batch, shift, scale):
    n_atoms = local_energies.shape[0]
    scale_p = jnp.zeros((TAB_PAD,), jnp.float32).at[: scale.shape[0]].set(scale)
    shift_p = jnp.zeros((TAB_PAD,), jnp.float32).at[: shift.shape[0]].set(shift)
    e2 = local_energies.reshape(-1, ROW)
    z2 = Z.reshape(-1, ROW)
    b2 = batch.reshape(-1, ROW)

    sc_kernel, nw = _make_sc_partials(n_atoms)
    partials = sc_kernel(e2, z2, b2, scale_p, shift_p)

    total = pl.pallas_call(
        _sum_partials_body,
        out_shape=jax.ShapeDtypeStruct((N_STRUCTURES,), jnp.float32),
    )(partials)
    return total


# speedup vs baseline: 1.9451x; 1.9451x over previous
import functools

import jax
import jax.numpy as jnp
from jax import lax
from jax.experimental import pallas as pl
from jax.experimental.pallas import tpu as pltpu
from jax.experimental.pallas import tpu_sc as plsc

N_STRUCTURES = 1024
TAB_PAD = 128
LANES = 16
ROW = 128
Z_BITS = 7


def _make_sc_partials(n_atoms):
    info = plsc.get_sparse_core_info()
    nc, ns = info.num_cores, info.num_subcores
    nw = nc * ns
    assert n_atoms % ROW == 0
    n_rows = n_atoms // ROW
    rows_w = n_rows // nw
    tail_rows = n_rows - rows_w * nw
    assert tail_rows <= nw

    chunk_rows = rows_w
    for cand in (100, 96, 90, 80, 78, 75, 65, 64, 60, 50, 40, 39, 32, 30,
                 26, 25, 20, 16, 15, 13, 10, 8, 6, 5, 4, 3, 2, 1):
        if rows_w % cand == 0:
            chunk_rows = cand
            break
    n_chunks = rows_w // chunk_rows
    vec_per_chunk = chunk_rows * (ROW // LANES)
    idx_len = ((chunk_rows + LANES - 1) // LANES) * LANES

    mesh = plsc.VectorSubcoreMesh(core_axis_name="c", subcore_axis_name="s")

    @functools.partial(
        pl.kernel,
        mesh=mesh,
        out_type=jax.ShapeDtypeStruct((nw, N_STRUCTURES), jnp.float32),
        compiler_params=pltpu.CompilerParams(
            needs_layout_passes=False, use_tc_tiling_on_sc=False
        ),
        scratch_types=[
            pltpu.VMEM((chunk_rows, ROW), jnp.float32),
            pltpu.VMEM((chunk_rows, ROW), jnp.int32),
            pltpu.VMEM((chunk_rows, ROW), jnp.float32),
            pltpu.VMEM((chunk_rows, ROW), jnp.int32),
            pltpu.VMEM((idx_len,), jnp.int32),
            pltpu.VMEM((idx_len,), jnp.int32),
            pltpu.VMEM((1, ROW), jnp.float32),
            pltpu.VMEM((1, ROW), jnp.int32),
            pltpu.VMEM((TAB_PAD,), jnp.float32),
            pltpu.VMEM((TAB_PAD,), jnp.float32),
            pltpu.VMEM((N_STRUCTURES * LANES,), jnp.float32),
            pltpu.VMEM((N_STRUCTURES,), jnp.float32),
            pltpu.SemaphoreType.DMA,
            pltpu.SemaphoreType.DMA,
            pltpu.SemaphoreType.DMA,
        ],
    )
    def sc_kernel(e_hbm, zb_hbm, scale_hbm, shift_hbm, out_hbm,
                  e0, x0, e1, x1, idx0, idx1, et, xt,
                  scale_v, shift_v, accx_v, acc_v, sem0, sem1, semt):
        sid = lax.axis_index("s")
        wid = sid * nc + lax.axis_index("c")
        row_base = wid * rows_w
        bufs = ((e0, x0), (e1, x1))
        idxs = (idx0, idx1)
        sems = (sem0, sem1)

        def start_chunk(ci, p):
            r0 = row_base + ci * chunk_rows
            for k in range(idx_len // LANES):
                idxs[p][pl.ds(k * LANES, LANES)] = (
                    lax.iota(jnp.int32, LANES) + (r0 + k * LANES)
                )
            rows = idxs[p].at[pl.ds(0, chunk_rows)]
            return (
                pltpu.async_copy(e_hbm.at[rows], bufs[p][0], sems[p]),
                pltpu.async_copy(zb_hbm.at[rows], bufs[p][1], sems[p]),
            )

        in_flight = start_chunk(0, 0)

        pltpu.sync_copy(scale_hbm, scale_v)
        pltpu.sync_copy(shift_hbm, shift_v)

        zeros16 = jnp.zeros((LANES,), jnp.float32)

        @plsc.parallel_loop(0, N_STRUCTURES * LANES, LANES, unroll=8)
        def _(off):
            accx_v[pl.ds(off, LANES)] = zeros16

        lane = lax.iota(jnp.int32, LANES)

        def process(e_v, x_v, nvec):
            @plsc.parallel_loop(0, nvec, 1, unroll=8)
            def _(j):
                r = j // (ROW // LANES)
                c = (j % (ROW // LANES)) * LANES
                e = e_v[r, pl.ds(c, LANES)]
                zb = x_v[r, pl.ds(c, LANES)]
                z = zb & ((1 << Z_BITS) - 1)
                b = lax.shift_right_logical(zb, Z_BITS)
                sc = plsc.load_gather(scale_v, [z])
                sh = plsc.load_gather(shift_v, [z])
                bank = (lane + j) & (LANES - 1)
                plsc.addupdate_scatter(
                    accx_v, [(b * LANES) + bank], e * sc + sh
                )

        for ci in range(n_chunks):
            p = ci % 2
            for d in in_flight:
                d.wait()
            if ci + 1 < n_chunks:
                in_flight = start_chunk(ci + 1, (ci + 1) % 2)
            e_v, x_v = bufs[p]
            process(e_v, x_v, vec_per_chunk)

        if tail_rows:
            @pl.when(wid < tail_rows)
            def _():
                tr = nw * rows_w + wid
                idx0[pl.ds(0, LANES)] = lax.iota(jnp.int32, LANES) + tr
                rows = idx0.at[pl.ds(0, 1)]
                d1 = pltpu.async_copy(e_hbm.at[rows], et, semt)
                d2 = pltpu.async_copy(zb_hbm.at[rows], xt, semt)
                d1.wait()
                d2.wait()
                process(et, xt, ROW // LANES)

        @plsc.parallel_loop(0, N_STRUCTURES // LANES, 1, unroll=4)
        def _(t):
            srow = (lane + t * LANES) * LANES
            tot = plsc.load_gather(accx_v, [srow])
            for jj in range(1, LANES):
                tot = tot + plsc.load_gather(accx_v, [srow + jj])
            acc_v[pl.ds(t * LANES, LANES)] = tot

        pltpu.sync_copy(acc_v, out_hbm.at[wid])

    return sc_kernel, nw


def _sum_partials_body(p_ref, o_ref):
    o_ref[:] = jnp.sum(p_ref[:], axis=0)


def kernel(local_energies, Z, batch, shift, scale):
    n_atoms = local_energies.shape[0]
    scale_p = jnp.zeros((TAB_PAD,), jnp.float32).at[: scale.shape[0]].set(scale)
    shift_p = jnp.zeros((TAB_PAD,), jnp.float32).at[: shift.shape[0]].set(shift)
    e2 = local_energies.reshape(-1, ROW)
    zb2 = ((batch << Z_BITS) | Z).reshape(-1, ROW)

    sc_kernel, nw = _make_sc_partials(n_atoms)
    partials = sc_kernel(e2, zb2, scale_p, shift_p)

    total = pl.pallas_call(
        _sum_partials_body,
        out_shape=jax.ShapeDtypeStruct((N_STRUCTURES,), jnp.float32),
    )(partials)
    return total


# speedup vs baseline: 2.0399x; 1.0488x over previous
import functools

import jax
import jax.numpy as jnp
from jax import lax
from jax.experimental import pallas as pl
from jax.experimental.pallas import tpu as pltpu
from jax.experimental.pallas import tpu_sc as plsc

N_STRUCTURES = 1024
TAB_PAD = 128
LANES = 16
ROW = 128
Z_BITS = 7


def _make_sc_partials(n_atoms, n_species):
    info = plsc.get_sparse_core_info()
    nc, ns = info.num_cores, info.num_subcores
    nw = nc * ns
    assert n_atoms % ROW == 0
    n_rows = n_atoms // ROW
    rows_w = n_rows // nw
    tail_rows = n_rows - rows_w * nw
    assert tail_rows <= nw

    chunk_rows = rows_w
    for cand in (100, 96, 90, 80, 78, 75, 65, 64, 60, 50, 40, 39, 32, 30,
                 26, 25, 20, 16, 15, 13, 10, 8, 6, 5, 4, 3, 2, 1):
        if rows_w % cand == 0:
            chunk_rows = cand
            break
    n_chunks = rows_w // chunk_rows
    vec_per_chunk = chunk_rows * (ROW // LANES)
    idx_len = ((chunk_rows + LANES - 1) // LANES) * LANES

    mesh = plsc.VectorSubcoreMesh(core_axis_name="c", subcore_axis_name="s")

    @functools.partial(
        pl.kernel,
        mesh=mesh,
        out_type=jax.ShapeDtypeStruct((nw, N_STRUCTURES), jnp.float32),
        compiler_params=pltpu.CompilerParams(
            needs_layout_passes=False, use_tc_tiling_on_sc=False
        ),
        scratch_types=[
            pltpu.VMEM((chunk_rows, ROW), jnp.float32),
            pltpu.VMEM((chunk_rows, ROW), jnp.int32),
            pltpu.VMEM((chunk_rows, ROW), jnp.float32),
            pltpu.VMEM((chunk_rows, ROW), jnp.int32),
            pltpu.VMEM((idx_len,), jnp.int32),
            pltpu.VMEM((idx_len,), jnp.int32),
            pltpu.VMEM((1, ROW), jnp.float32),
            pltpu.VMEM((1, ROW), jnp.int32),
            pltpu.VMEM((TAB_PAD,), jnp.float32),
            pltpu.VMEM((TAB_PAD,), jnp.float32),
            pltpu.VMEM((N_STRUCTURES * LANES,), jnp.float32),
            pltpu.VMEM((N_STRUCTURES,), jnp.float32),
            pltpu.SemaphoreType.DMA,
            pltpu.SemaphoreType.DMA,
            pltpu.SemaphoreType.DMA,
        ],
    )
    def sc_kernel(e_hbm, zb_hbm, scale_hbm, shift_hbm, out_hbm,
                  e0, x0, e1, x1, idx0, idx1, et, xt,
                  scale_v, shift_v, accx_v, acc_v, sem0, sem1, semt):
        sid = lax.axis_index("s")
        wid = sid * nc + lax.axis_index("c")
        row_base = wid * rows_w
        bufs = ((e0, x0), (e1, x1))
        idxs = (idx0, idx1)
        sems = (sem0, sem1)

        def start_chunk(ci, p):
            r0 = row_base + ci * chunk_rows
            for k in range(idx_len // LANES):
                idxs[p][pl.ds(k * LANES, LANES)] = (
                    lax.iota(jnp.int32, LANES) + (r0 + k * LANES)
                )
            rows = idxs[p].at[pl.ds(0, chunk_rows)]
            return (
                pltpu.async_copy(e_hbm.at[rows], bufs[p][0], sems[p]),
                pltpu.async_copy(zb_hbm.at[rows], bufs[p][1], sems[p]),
            )

        in_flight = start_chunk(0, 0)

        pltpu.sync_copy(scale_hbm, scale_v.at[pl.ds(0, n_species)])
        pltpu.sync_copy(shift_hbm, shift_v.at[pl.ds(0, n_species)])

        zeros16 = jnp.zeros((LANES,), jnp.float32)

        @plsc.parallel_loop(0, N_STRUCTURES * LANES, LANES, unroll=8)
        def _(off):
            accx_v[pl.ds(off, LANES)] = zeros16

        lane = lax.iota(jnp.int32, LANES)

        def process(e_v, x_v, nvec):
            @plsc.parallel_loop(0, nvec, 1, unroll=8)
            def _(j):
                r = j // (ROW // LANES)
                c = (j % (ROW // LANES)) * LANES
                e = e_v[r, pl.ds(c, LANES)]
                zb = x_v[r, pl.ds(c, LANES)]
                z = zb & ((1 << Z_BITS) - 1)
                b = lax.shift_right_logical(zb, Z_BITS)
                sc = plsc.load_gather(scale_v, [z])
                sh = plsc.load_gather(shift_v, [z])
                bank = (lane + j) & (LANES - 1)
                plsc.addupdate_scatter(
                    accx_v, [(b * LANES) + bank], e * sc + sh
                )

        for ci in range(n_chunks):
            p = ci % 2
            for d in in_flight:
                d.wait()
            if ci + 1 < n_chunks:
                in_flight = start_chunk(ci + 1, (ci + 1) % 2)
            e_v, x_v = bufs[p]
            process(e_v, x_v, vec_per_chunk)

        if tail_rows:
            @pl.when(wid < tail_rows)
            def _():
                tr = nw * rows_w + wid
                idx0[pl.ds(0, LANES)] = lax.iota(jnp.int32, LANES) + tr
                rows = idx0.at[pl.ds(0, 1)]
                d1 = pltpu.async_copy(e_hbm.at[rows], et, semt)
                d2 = pltpu.async_copy(zb_hbm.at[rows], xt, semt)
                d1.wait()
                d2.wait()
                process(et, xt, ROW // LANES)

        @plsc.parallel_loop(0, N_STRUCTURES // LANES, 1, unroll=4)
        def _(t):
            srow = (lane + t * LANES) * LANES
            tot = plsc.load_gather(accx_v, [srow])
            for jj in range(1, LANES):
                tot = tot + plsc.load_gather(accx_v, [srow + jj])
            acc_v[pl.ds(t * LANES, LANES)] = tot

        pltpu.sync_copy(acc_v, out_hbm.at[wid])

    return sc_kernel, nw


def _sum_partials_body(p_ref, o_ref):
    o_ref[:] = jnp.sum(p_ref[:], axis=0)


def kernel(local_energies, Z, batch, shift, scale):
    n_atoms = local_energies.shape[0]
    e2 = local_energies.reshape(-1, ROW)
    zb2 = ((batch << Z_BITS) | Z).reshape(-1, ROW)

    sc_kernel, nw = _make_sc_partials(n_atoms, scale.shape[0])
    partials = sc_kernel(e2, zb2, scale, shift)

    total = pl.pallas_call(
        _sum_partials_body,
        out_shape=jax.ShapeDtypeStruct((N_STRUCTURES,), jnp.float32),
    )(partials)
    return total


# speedup vs baseline: 2.0816x; 1.0205x over previous
import functools

import jax
import jax.numpy as jnp
from jax import lax
from jax.experimental import pallas as pl
from jax.experimental.pallas import tpu as pltpu
from jax.experimental.pallas import tpu_sc as plsc

N_STRUCTURES = 1024
TAB_PAD = 128
LANES = 16
ROW = 128
Z_BITS = 7


def _make_sc_partials(n_atoms, n_species):
    info = plsc.get_sparse_core_info()
    nc, ns = info.num_cores, info.num_subcores
    nw = nc * ns
    assert n_atoms % ROW == 0
    n_rows = n_atoms // ROW
    rows_w = n_rows // nw
    tail_rows = n_rows - rows_w * nw
    assert tail_rows <= nw

    chunk_rows = rows_w
    for cand in (100, 96, 90, 80, 78, 75, 65, 64, 60, 50, 40, 39, 32, 30,
                 26, 25, 20, 16, 15, 13, 10, 8, 6, 5, 4, 3, 2, 1):
        if rows_w % cand == 0:
            chunk_rows = cand
            break
    n_chunks = rows_w // chunk_rows
    vec_per_chunk = chunk_rows * (ROW // LANES)
    idx_len = ((chunk_rows + LANES - 1) // LANES) * LANES

    mesh = plsc.VectorSubcoreMesh(core_axis_name="c", subcore_axis_name="s")

    @functools.partial(
        pl.kernel,
        mesh=mesh,
        out_type=jax.ShapeDtypeStruct((nw, N_STRUCTURES), jnp.float32),
        compiler_params=pltpu.CompilerParams(
            needs_layout_passes=False, use_tc_tiling_on_sc=False
        ),
        scratch_types=[
            pltpu.VMEM((chunk_rows, ROW), jnp.float32),
            pltpu.VMEM((chunk_rows, ROW), jnp.int32),
            pltpu.VMEM((chunk_rows, ROW), jnp.int32),
            pltpu.VMEM((chunk_rows, ROW), jnp.float32),
            pltpu.VMEM((chunk_rows, ROW), jnp.int32),
            pltpu.VMEM((chunk_rows, ROW), jnp.int32),
            pltpu.VMEM((idx_len,), jnp.int32),
            pltpu.VMEM((idx_len,), jnp.int32),
            pltpu.VMEM((1, ROW), jnp.float32),
            pltpu.VMEM((1, ROW), jnp.int32),
            pltpu.VMEM((1, ROW), jnp.int32),
            pltpu.VMEM((TAB_PAD,), jnp.float32),
            pltpu.VMEM((TAB_PAD,), jnp.float32),
            pltpu.VMEM((N_STRUCTURES * LANES,), jnp.float32),
            pltpu.VMEM((N_STRUCTURES,), jnp.float32),
            pltpu.SemaphoreType.DMA,
            pltpu.SemaphoreType.DMA,
            pltpu.SemaphoreType.DMA,
        ],
    )
    def sc_kernel(e_hbm, z_hbm, b_hbm, scale_hbm, shift_hbm, out_hbm,
                  e0, z0, b0, e1, z1, b1, idx0, idx1, et, zt, bt,
                  scale_v, shift_v, accx_v, acc_v, sem0, sem1, semt):
        sid = lax.axis_index("s")
        wid = sid * nc + lax.axis_index("c")
        row_base = wid * rows_w
        bufs = ((e0, z0, b0), (e1, z1, b1))
        idxs = (idx0, idx1)
        sems = (sem0, sem1)

        def start_chunk(ci, p):
            r0 = row_base + ci * chunk_rows
            for k in range(idx_len // LANES):
                idxs[p][pl.ds(k * LANES, LANES)] = (
                    lax.iota(jnp.int32, LANES) + (r0 + k * LANES)
                )
            rows = idxs[p].at[pl.ds(0, chunk_rows)]
            return (
                pltpu.async_copy(e_hbm.at[rows], bufs[p][0], sems[p]),
                pltpu.async_copy(z_hbm.at[rows], bufs[p][1], sems[p]),
                pltpu.async_copy(b_hbm.at[rows], bufs[p][2], sems[p]),
            )

        in_flight = start_chunk(0, 0)

        pltpu.sync_copy(scale_hbm, scale_v.at[pl.ds(0, n_species)])
        pltpu.sync_copy(shift_hbm, shift_v.at[pl.ds(0, n_species)])

        zeros16 = jnp.zeros((LANES,), jnp.float32)

        @plsc.parallel_loop(0, N_STRUCTURES * LANES, LANES, unroll=8)
        def _(off):
            accx_v[pl.ds(off, LANES)] = zeros16

        lane = lax.iota(jnp.int32, LANES)

        def process(e_v, z_v, b_v, nvec):
            @plsc.parallel_loop(0, nvec, 1, unroll=8)
            def _(j):
                r = j // (ROW // LANES)
                c = (j % (ROW // LANES)) * LANES
                e = e_v[r, pl.ds(c, LANES)]
                z = z_v[r, pl.ds(c, LANES)]
                b = b_v[r, pl.ds(c, LANES)]
                sc = plsc.load_gather(scale_v, [z])
                sh = plsc.load_gather(shift_v, [z])
                bank = (lane + j) & (LANES - 1)
                plsc.addupdate_scatter(
                    accx_v, [(b * LANES) + bank], e * sc + sh
                )

        for ci in range(n_chunks):
            p = ci % 2
            for d in in_flight:
                d.wait()
            if ci + 1 < n_chunks:
                in_flight = start_chunk(ci + 1, (ci + 1) % 2)
            e_v, z_v, b_v = bufs[p]
            process(e_v, z_v, b_v, vec_per_chunk)

        if tail_rows:
            @pl.when(wid < tail_rows)
            def _():
                tr = nw * rows_w + wid
                idx0[pl.ds(0, LANES)] = lax.iota(jnp.int32, LANES) + tr
                rows = idx0.at[pl.ds(0, 1)]
                d1 = pltpu.async_copy(e_hbm.at[rows], et, semt)
                d2 = pltpu.async_copy(z_hbm.at[rows], zt, semt)
                d3 = pltpu.async_copy(b_hbm.at[rows], bt, semt)
                d1.wait()
                d2.wait()
                d3.wait()
                process(et, zt, bt, ROW // LANES)

        @plsc.parallel_loop(0, N_STRUCTURES // LANES, 1, unroll=4)
        def _(t):
            srow = (lane + t * LANES) * LANES
            tot = plsc.load_gather(accx_v, [srow])
            for jj in range(1, LANES):
                tot = tot + plsc.load_gather(accx_v, [srow + jj])
            acc_v[pl.ds(t * LANES, LANES)] = tot

        pltpu.sync_copy(acc_v, out_hbm.at[wid])

    return sc_kernel, nw


def _sum_partials_body(p_ref, o_ref):
    o_ref[:] = jnp.sum(p_ref[:], axis=0)


def kernel(local_energies, Z, batch, shift, scale):
    n_atoms = local_energies.shape[0]
    e2 = local_energies.reshape(-1, ROW)
    z2 = Z.reshape(-1, ROW)
    b2 = batch.reshape(-1, ROW)

    sc_kernel, nw = _make_sc_partials(n_atoms, scale.shape[0])
    partials = sc_kernel(e2, z2, b2, scale, shift)

    total = pl.pallas_call(
        _sum_partials_body,
        out_shape=jax.ShapeDtypeStruct((N_STRUCTURES,), jnp.float32),
    )(partials)
    return total


# speedup vs baseline: 2.0871x; 1.0026x over previous
import functools

import jax
import jax.numpy as jnp
from jax import lax
from jax.experimental import pallas as pl
from jax.experimental.pallas import tpu as pltpu
from jax.experimental.pallas import tpu_sc as plsc

N_STRUCTURES = 1024
TAB_PAD = 128
LANES = 16
ROW = 256
Z_BITS = 7


def _make_sc_partials(n_atoms, n_species):
    info = plsc.get_sparse_core_info()
    nc, ns = info.num_cores, info.num_subcores
    nw = nc * ns
    assert n_atoms % ROW == 0
    n_rows = n_atoms // ROW
    rows_w = n_rows // nw
    tail_rows = n_rows - rows_w * nw
    assert tail_rows <= nw

    chunk_rows = rows_w
    for cand in (100, 96, 90, 80, 78, 75, 65, 64, 60, 50, 40, 39, 32, 30,
                 26, 25, 20, 16, 15, 13, 10, 8, 6, 5, 4, 3, 2, 1):
        if rows_w % cand == 0:
            chunk_rows = cand
            break
    n_chunks = rows_w // chunk_rows
    vec_per_chunk = chunk_rows * (ROW // LANES)
    idx_len = ((chunk_rows + LANES - 1) // LANES) * LANES

    mesh = plsc.VectorSubcoreMesh(core_axis_name="c", subcore_axis_name="s")

    @functools.partial(
        pl.kernel,
        mesh=mesh,
        out_type=jax.ShapeDtypeStruct((nw, N_STRUCTURES), jnp.float32),
        compiler_params=pltpu.CompilerParams(
            needs_layout_passes=False, use_tc_tiling_on_sc=False
        ),
        scratch_types=[
            pltpu.VMEM((chunk_rows, ROW), jnp.float32),
            pltpu.VMEM((chunk_rows, ROW), jnp.int32),
            pltpu.VMEM((chunk_rows, ROW), jnp.int32),
            pltpu.VMEM((chunk_rows, ROW), jnp.float32),
            pltpu.VMEM((chunk_rows, ROW), jnp.int32),
            pltpu.VMEM((chunk_rows, ROW), jnp.int32),
            pltpu.VMEM((idx_len,), jnp.int32),
            pltpu.VMEM((idx_len,), jnp.int32),
            pltpu.VMEM((1, ROW), jnp.float32),
            pltpu.VMEM((1, ROW), jnp.int32),
            pltpu.VMEM((1, ROW), jnp.int32),
            pltpu.VMEM((TAB_PAD,), jnp.float32),
            pltpu.VMEM((TAB_PAD,), jnp.float32),
            pltpu.VMEM((N_STRUCTURES * LANES,), jnp.float32),
            pltpu.VMEM((N_STRUCTURES,), jnp.float32),
            pltpu.SemaphoreType.DMA,
            pltpu.SemaphoreType.DMA,
            pltpu.SemaphoreType.DMA,
        ],
    )
    def sc_kernel(e_hbm, z_hbm, b_hbm, scale_hbm, shift_hbm, out_hbm,
                  e0, z0, b0, e1, z1, b1, idx0, idx1, et, zt, bt,
                  scale_v, shift_v, accx_v, acc_v, sem0, sem1, semt):
        sid = lax.axis_index("s")
        wid = sid * nc + lax.axis_index("c")
        row_base = wid * rows_w
        bufs = ((e0, z0, b0), (e1, z1, b1))
        idxs = (idx0, idx1)
        sems = (sem0, sem1)

        def start_chunk(ci, p):
            r0 = row_base + ci * chunk_rows
            for k in range(idx_len // LANES):
                idxs[p][pl.ds(k * LANES, LANES)] = (
                    lax.iota(jnp.int32, LANES) + (r0 + k * LANES)
                )
            rows = idxs[p].at[pl.ds(0, chunk_rows)]
            return (
                pltpu.async_copy(e_hbm.at[rows], bufs[p][0], sems[p]),
                pltpu.async_copy(z_hbm.at[rows], bufs[p][1], sems[p]),
                pltpu.async_copy(b_hbm.at[rows], bufs[p][2], sems[p]),
            )

        in_flight = start_chunk(0, 0)

        pltpu.sync_copy(scale_hbm, scale_v.at[pl.ds(0, n_species)])
        pltpu.sync_copy(shift_hbm, shift_v.at[pl.ds(0, n_species)])

        zeros16 = jnp.zeros((LANES,), jnp.float32)

        @plsc.parallel_loop(0, N_STRUCTURES * LANES, LANES, unroll=8)
        def _(off):
            accx_v[pl.ds(off, LANES)] = zeros16

        lane = lax.iota(jnp.int32, LANES)

        def process(e_v, z_v, b_v, nvec):
            @plsc.parallel_loop(0, nvec, 1, unroll=8)
            def _(j):
                r = j // (ROW // LANES)
                c = (j % (ROW // LANES)) * LANES
                e = e_v[r, pl.ds(c, LANES)]
                z = z_v[r, pl.ds(c, LANES)]
                b = b_v[r, pl.ds(c, LANES)]
                sc = plsc.load_gather(scale_v, [z])
                sh = plsc.load_gather(shift_v, [z])
                bank = (lane + j) & (LANES - 1)
                plsc.addupdate_scatter(
                    accx_v, [(b * LANES) + bank], e * sc + sh
                )

        for ci in range(n_chunks):
            p = ci % 2
            for d in in_flight:
                d.wait()
            if ci + 1 < n_chunks:
                in_flight = start_chunk(ci + 1, (ci + 1) % 2)
            e_v, z_v, b_v = bufs[p]
            process(e_v, z_v, b_v, vec_per_chunk)

        if tail_rows:
            @pl.when(wid < tail_rows)
            def _():
                tr = nw * rows_w + wid
                idx0[pl.ds(0, LANES)] = lax.iota(jnp.int32, LANES) + tr
                rows = idx0.at[pl.ds(0, 1)]
                d1 = pltpu.async_copy(e_hbm.at[rows], et, semt)
                d2 = pltpu.async_copy(z_hbm.at[rows], zt, semt)
                d3 = pltpu.async_copy(b_hbm.at[rows], bt, semt)
                d1.wait()
                d2.wait()
                d3.wait()
                process(et, zt, bt, ROW // LANES)

        @plsc.parallel_loop(0, N_STRUCTURES // LANES, 1, unroll=4)
        def _(t):
            srow = (lane + t * LANES) * LANES
            tot = plsc.load_gather(accx_v, [srow])
            for jj in range(1, LANES):
                tot = tot + plsc.load_gather(accx_v, [srow + jj])
            acc_v[pl.ds(t * LANES, LANES)] = tot

        pltpu.sync_copy(acc_v, out_hbm.at[wid])

    return sc_kernel, nw


def _sum_partials_body(p_ref, o_ref):
    o_ref[:] = jnp.sum(p_ref[:], axis=0)


def kernel(local_energies, Z, batch, shift, scale):
    n_atoms = local_energies.shape[0]
    e2 = local_energies.reshape(-1, ROW)
    z2 = Z.reshape(-1, ROW)
    b2 = batch.reshape(-1, ROW)

    sc_kernel, nw = _make_sc_partials(n_atoms, scale.shape[0])
    partials = sc_kernel(e2, z2, b2, scale, shift)

    total = pl.pallas_call(
        _sum_partials_body,
        out_shape=jax.ShapeDtypeStruct((N_STRUCTURES,), jnp.float32),
    )(partials)
    return total
